# baseline (speedup 1.0000x reference)
import jax
import jax.numpy as jnp
from jax.experimental import pallas as pl


def _pick_block(t: int, target: int = 2048) -> int:
    for cand in (target, 1024, 512, 256, 128, 64, 32, 16, 8):
        if t % cand == 0 and cand <= t:
            return cand
    return t


def kernel(series, W1, b1, W2, b2, codebook, g_noise):
    B, F, T = series.shape
    H = W1.shape[0]
    C = W2.shape[0]
    G = F // codebook.shape[2]
    CG = C // G
    FG = F // G

    blk = _pick_block(T)

    gnT = g_noise.T
    nb = T // blk
    cbT = codebook.reshape(G, CG, FG).transpose(0, 2, 1).reshape(G * FG, CG)
    b1c = b1.reshape(H, 1)
    b2c = b2.reshape(C, 1)

    def kern(s_ref, w1_ref, b1_ref, w2_ref, b2_ref, cb_ref, gn_ref,
             q_ref, idx_ref):
        s = s_ref[0]
        h = jnp.dot(w1_ref[...], s, preferred_element_type=jnp.float32)
        h = jnp.maximum(h + b1_ref[...], 0.0)
        z = jnp.dot(w2_ref[...], h, preferred_element_type=jnp.float32)
        zb = z + b2_ref[...]
        gn_d = jnp.transpose(gn_ref[...].T.reshape(blk, G, CG), (1, 0, 2))
        idx_rows = []
        for g in range(G):
            l = zb[g * CG:(g + 1) * CG, :] + gn_d[g].T
            m = jnp.max(l, axis=0, keepdims=True)
            iota = jax.lax.broadcasted_iota(jnp.int32, (CG, blk), 0)
            idxg = jnp.min(jnp.where(l == m, iota, CG), axis=0, keepdims=True)
            oh = (iota == idxg).astype(jnp.float32)
            q_ref[0, g * FG:(g + 1) * FG, :] = jnp.dot(
                cb_ref[g * FG:(g + 1) * FG, :], oh,
                preferred_element_type=jnp.float32)
            idx_rows.append(idxg)
        idx_ref[0] = jnp.concatenate(idx_rows, axis=0)

    q, idx = pl.pallas_call(
        kern,
        grid=(B, T // blk),
        in_specs=[
            pl.BlockSpec((1, F, blk), lambda b, t: (b, 0, t)),
            pl.BlockSpec((H, F), lambda b, t: (0, 0)),
            pl.BlockSpec((H, 1), lambda b, t: (0, 0)),
            pl.BlockSpec((C, H), lambda b, t: (0, 0)),
            pl.BlockSpec((C, 1), lambda b, t: (0, 0)),
            pl.BlockSpec((G * FG, CG), lambda b, t: (0, 0)),
            pl.BlockSpec((CG, G * blk), lambda b, t, _nb=nb: (0, b * _nb + t)),
        ],
        out_specs=[
            pl.BlockSpec((1, F, blk), lambda b, t: (b, 0, t)),
            pl.BlockSpec((1, G, blk), lambda b, t: (b, 0, t)),
        ],
        out_shape=[
            jax.ShapeDtypeStruct((B, F, T), jnp.float32),
            jax.ShapeDtypeStruct((B, G, T), jnp.int32),
        ],
    )(series, W1, b1c, W2, b2c, cbT, gnT)
    return q, idx.transpose(0, 2, 1)


# speedup vs baseline: 1.0008x; 1.0008x over previous
import jax
import jax.numpy as jnp
from jax.experimental import pallas as pl


def _pick_block(t: int, target: int = 1024) -> int:
    for cand in (target, 512, 256, 128, 64, 32, 16, 8):
        if t % cand == 0 and cand <= t:
            return cand
    return t


def kernel(series, W1, b1, W2, b2, codebook, g_noise):
    B, F, T = series.shape
    H = W1.shape[0]
    C = W2.shape[0]
    G = F // codebook.shape[2]
    CG = C // G
    FG = F // G

    blk = _pick_block(T)

    gnT = g_noise.T
    nb = T // blk
    cbT = codebook.reshape(G, CG, FG).transpose(0, 2, 1).reshape(G * FG, CG)
    b1c = b1.reshape(H, 1)
    b2c = b2.reshape(C, 1)

    def kern(s_ref, w1_ref, b1_ref, w2_ref, b2_ref, cb_ref, gn_ref,
             q_ref, idx_ref):
        s = s_ref[0]
        h = jnp.dot(w1_ref[...], s, preferred_element_type=jnp.float32)
        h = jnp.maximum(h + b1_ref[...], 0.0)
        z = jnp.dot(w2_ref[...], h, preferred_element_type=jnp.float32)
        zb = z + b2_ref[...]
        gn_d = jnp.transpose(gn_ref[...].T.reshape(blk, G, CG), (1, 0, 2))
        idx_rows = []
        for g in range(G):
            l = zb[g * CG:(g + 1) * CG, :] + gn_d[g].T
            m = jnp.max(l, axis=0, keepdims=True)
            iota = jax.lax.broadcasted_iota(jnp.int32, (CG, blk), 0)
            idxg = jnp.min(jnp.where(l == m, iota, CG), axis=0, keepdims=True)
            oh = (iota == idxg).astype(jnp.float32)
            q_ref[0, g * FG:(g + 1) * FG, :] = jnp.dot(
                cb_ref[g * FG:(g + 1) * FG, :], oh,
                preferred_element_type=jnp.float32)
            idx_rows.append(idxg)
        idx_ref[0] = jnp.concatenate(idx_rows, axis=0)

    q, idx = pl.pallas_call(
        kern,
        grid=(B, T // blk),
        in_specs=[
            pl.BlockSpec((1, F, blk), lambda b, t: (b, 0, t)),
            pl.BlockSpec((H, F), lambda b, t: (0, 0)),
            pl.BlockSpec((H, 1), lambda b, t: (0, 0)),
            pl.BlockSpec((C, H), lambda b, t: (0, 0)),
            pl.BlockSpec((C, 1), lambda b, t: (0, 0)),
            pl.BlockSpec((G * FG, CG), lambda b, t: (0, 0)),
            pl.BlockSpec((CG, G * blk), lambda b, t, _nb=nb: (0, b * _nb + t)),
        ],
        out_specs=[
            pl.BlockSpec((1, F, blk), lambda b, t: (b, 0, t)),
            pl.BlockSpec((1, G, blk), lambda b, t: (b, 0, t)),
        ],
        out_shape=[
            jax.ShapeDtypeStruct((B, F, T), jnp.float32),
            jax.ShapeDtypeStruct((B, G, T), jnp.int32),
        ],
    )(series, W1, b1c, W2, b2c, cbT, gnT)
    return q, idx.transpose(0, 2, 1)


# speedup vs baseline: 1.0398x; 1.0389x over previous
import jax
import jax.numpy as jnp
from jax.experimental import pallas as pl


def _pick_block(t: int, target: int = 1024) -> int:
    for cand in (target, 512, 256, 128, 64, 32, 16, 8):
        if t % cand == 0 and cand <= t:
            return cand
    return t


def kernel(series, W1, b1, W2, b2, codebook, g_noise):
    B, F, T = series.shape
    H = W1.shape[0]
    C = W2.shape[0]
    G = F // codebook.shape[2]
    CG = C // G
    FG = F // G

    blk = _pick_block(T)

    gnT = g_noise.T
    nb = T // blk
    cb2 = codebook.reshape(C, FG)

    def kern(s_ref, w1_ref, b1_ref, w2_ref, b2_ref, cb_ref, gn_ref,
             q_ref, idx_ref):
        s = s_ref[0]
        h = jnp.dot(w1_ref[...], s, preferred_element_type=jnp.float32)
        h = jnp.maximum(h + b1_ref[...].reshape(1, H).T, 0.0)
        z = jnp.dot(w2_ref[...], h, preferred_element_type=jnp.float32)
        zb = z + b2_ref[...].reshape(1, C).T
        gn_d = jnp.transpose(gn_ref[...].T.reshape(blk, G, CG), (1, 0, 2))
        idx_rows = []
        for g in range(G):
            l = zb[g * CG:(g + 1) * CG, :] + gn_d[g].T
            m = jnp.max(l, axis=0, keepdims=True)
            iota = jax.lax.broadcasted_iota(jnp.int32, (CG, blk), 0)
            idxg = jnp.min(jnp.where(l == m, iota, CG), axis=0, keepdims=True)
            oh = (iota == idxg).astype(jnp.float32)
            q_ref[0, g * FG:(g + 1) * FG, :] = jax.lax.dot_general(
                cb_ref[g * CG:(g + 1) * CG, :], oh, (((0,), (0,)), ((), ())),
                preferred_element_type=jnp.float32)
            idx_rows.append(idxg)
        idx_ref[0] = jnp.concatenate(idx_rows, axis=0)

    q, idx = pl.pallas_call(
        kern,
        grid=(B, T // blk),
        in_specs=[
            pl.BlockSpec((1, F, blk), lambda b, t: (b, 0, t)),
            pl.BlockSpec((H, F), lambda b, t: (0, 0)),
            pl.BlockSpec((H,), lambda b, t: (0,)),
            pl.BlockSpec((C, H), lambda b, t: (0, 0)),
            pl.BlockSpec((C,), lambda b, t: (0,)),
            pl.BlockSpec((C, FG), lambda b, t: (0, 0)),
            pl.BlockSpec((CG, G * blk), lambda b, t, _nb=nb: (0, b * _nb + t)),
        ],
        out_specs=[
            pl.BlockSpec((1, F, blk), lambda b, t: (b, 0, t)),
            pl.BlockSpec((1, G, blk), lambda b, t: (b, 0, t)),
        ],
        out_shape=[
            jax.ShapeDtypeStruct((B, F, T), jnp.float32),
            jax.ShapeDtypeStruct((B, G, T), jnp.int32),
        ],
    )(series, W1, b1, W2, b2, cb2, gnT)
    return q, idx.transpose(0, 2, 1)
